# baseline (speedup 1.0000x reference)
import functools

import jax
import jax.numpy as jnp
from jax import lax
from jax.experimental import pallas as pl
from jax.experimental.pallas import tpu as pltpu

_L = 21
_P = 32
_NG = 96
_T = _P + _NG
_TOPK = 2000
_NMS = 300
_NEG_INF = -1e9
_CH, _CW = 7, 7

_R = 160
_NPAD = _R * 128
_R3 = 3


_NC = 2048
_RC = 16


def _sel_kernel(scores_ref, anch_ref, delt_ref, gtb_ref, gtl_ref,
                roi_ref, dout_ref, lout_ref,
                pir_ref, rowoff_ref, vals_ref, cmp_ref, cco_ref,
                *, n_real, n_gt):
    f32 = jnp.float32
    s = scores_ref[0]
    a0 = anch_ref[0, 0]; a1 = anch_ref[0, 1]
    a2 = anch_ref[0, 2]; a3 = anch_ref[0, 3]
    d0 = delt_ref[0, 0]; d1 = delt_ref[0, 1]
    d2 = delt_ref[0, 2]; d3 = delt_ref[0, 3]

    aw = a3 - a1
    ah = a2 - a0
    acx = a1 + 0.5 * aw
    acy = a0 + 0.5 * ah
    bw = jnp.exp(d3) * aw
    bh = jnp.exp(d2) * ah
    bcx = d1 * aw + acx
    bcy = d0 * ah + acy
    y1 = bcy - 0.5 * bh
    x1 = bcx - 0.5 * bw
    y2 = bh + y1
    x2 = bw + x1
    y1 = jnp.clip(y1, 0.0, 1.0); x1 = jnp.clip(x1, 0.0, 1.0)
    y2 = jnp.clip(y2, 0.0, 1.0); x2 = jnp.clip(x2, 0.0, 1.0)
    area = jnp.maximum(y2 - y1, 0.0) * jnp.maximum(x2 - x1, 0.0)

    bits = lax.bitcast_convert_type(s, jnp.int32)

    def _bs_body(_, lohi):
        lo, hi = lohi
        mid = lo + lax.div(hi - lo, jnp.int32(2))
        cnt = jnp.sum(jnp.where(bits >= mid, 1.0, 0.0))
        ok = cnt >= float(_TOPK)
        return (jnp.where(ok, mid, lo), jnp.where(ok, hi, mid))

    lo0 = jnp.int32(0)
    hi0 = jnp.int32(2139095041)
    tau, _ = lax.fori_loop(0, 31, _bs_body, (lo0, hi0))

    cnt_gt = jnp.sum(jnp.where(bits > tau, 1.0, 0.0))
    m_needed = float(_TOPK) - cnt_gt
    tie = (bits == tau)
    tie_f = jnp.where(tie, 1.0, 0.0)
    col_i = lax.broadcasted_iota(jnp.int32, (128, 128), 0)
    col_j = lax.broadcasted_iota(jnp.int32, (128, 128), 1)
    slt_c = jnp.where(col_i < col_j, 1.0, 0.0).astype(f32)
    row_i = lax.broadcasted_iota(jnp.int32, (_R, _R), 0)
    row_j = lax.broadcasted_iota(jnp.int32, (_R, _R), 1)
    slt_r = jnp.where(row_j < row_i, 1.0, 0.0).astype(f32)
    within = lax.dot_general(tie_f, slt_c, (((1,), (0,)), ((), ())),
                             preferred_element_type=f32)
    rowsum = jnp.sum(tie_f, axis=1, keepdims=True)
    before = lax.dot_general(slt_r, rowsum, (((1,), (0,)), ((), ())),
                             preferred_element_type=f32)
    prefix = within + before
    eligible = (bits > tau) | (tie & (prefix < m_needed))

    flat = (lax.broadcasted_iota(jnp.int32, (_R, 128), 0) * 128
            + lax.broadcasted_iota(jnp.int32, (_R, 128), 1)).astype(f32)
    flat3 = (lax.broadcasted_iota(jnp.int32, (_R3, 128), 0) * 128
             + lax.broadcasted_iota(jnp.int32, (_R3, 128), 1)).astype(f32)

    elig_f = jnp.where(eligible, 1.0, 0.0)
    within_e = lax.dot_general(elig_f, slt_c, (((1,), (0,)), ((), ())),
                               preferred_element_type=f32)
    rowsum_e = jnp.sum(elig_f, axis=1, keepdims=True)
    before_e = lax.dot_general(slt_r, rowsum_e, (((1,), (0,)), ((), ())),
                               preferred_element_type=f32)

    pir_ref[...] = jnp.where(eligible, within_e, 1e9)
    rowoff_ref[...] = jnp.broadcast_to(before_e, (_R, 128))
    vals_ref[0] = y1; vals_ref[1] = x1; vals_ref[2] = y2; vals_ref[3] = x2
    vals_ref[4] = area; vals_ref[5] = s; vals_ref[6] = flat
    vals_ref[7] = jnp.zeros((_R, 128), f32)

    tgt128 = lax.broadcasted_iota(jnp.int32, (128, 128), 0).astype(f32)

    cmp_ref[...] = jnp.zeros((_NC + 128, 8), f32)

    def _comp_body(r, _):
        pir = pir_ref[pl.ds(r, 1), :]
        p = rowoff_ref[pl.ds(r, 1), :][0, 0].astype(jnp.int32)
        tr0 = lax.div(p, jnp.int32(128))
        off = (p - tr0 * 128).astype(f32)
        tloc = pir + off
        T0 = jnp.where(tgt128 == tloc, 1.0, 0.0)
        T1 = jnp.where(tgt128 == tloc - 128.0, 1.0, 0.0)
        vrow = vals_ref[:, pl.ds(r, 1), :].reshape(8, 128)
        V = vrow.T
        C0 = lax.dot_general(T0, V, (((1,), (0,)), ((), ())),
                             preferred_element_type=f32,
                             precision=lax.Precision.HIGHEST)
        C1 = lax.dot_general(T1, V, (((1,), (0,)), ((), ())),
                             preferred_element_type=f32,
                             precision=lax.Precision.HIGHEST)
        b0 = tr0 * 128
        cmp_ref[pl.ds(b0, 128), :] += C0
        cmp_ref[pl.ds(b0 + 128, 128), :] += C1
        return 0

    lax.fori_loop(0, _R, _comp_body, 0)

    for b in range(_RC):
        blk = cmp_ref[pl.ds(b * 128, 128), :]
        cco_ref[:, b, :] = blk.T

    cposf = (lax.broadcasted_iota(jnp.int32, (_RC, 128), 0) * 128
             + lax.broadcasted_iota(jnp.int32, (_RC, 128), 1)).astype(f32)
    valid_c = cposf < float(_TOPK)
    cy1 = cco_ref[0]; cx1 = cco_ref[1]; cy2 = cco_ref[2]; cx2 = cco_ref[3]
    carea = cco_ref[4]
    csc0 = jnp.where(valid_c, cco_ref[5], _NEG_INF)
    cflat = jnp.where(valid_c, cco_ref[6], 2e9 + cposf)

    def _nms_body(k, carry):
        sc, ny1, nx1, ny2, nx2 = carry
        m = jnp.max(sc)
        idx = jnp.min(jnp.where(sc == m, cflat, 4e9))
        cp = jnp.min(jnp.where(cflat == idx, cposf, 1e9)).astype(jnp.int32)
        row = cmp_ref[pl.ds(cp, 1), :]
        by1 = row[0, 0]; bx1 = row[0, 1]; by2 = row[0, 2]; bx2 = row[0, 3]
        barea = row[0, 4]
        valid = m > _NEG_INF * 0.5
        ry1 = jnp.where(valid, by1, 0.0); rx1 = jnp.where(valid, bx1, 0.0)
        ry2 = jnp.where(valid, by2, 0.0); rx2 = jnp.where(valid, bx2, 0.0)
        koh = jnp.where(flat3 == k.astype(f32), 1.0, 0.0)
        ny1 = ny1 + koh * ry1; nx1 = nx1 + koh * rx1
        ny2 = ny2 + koh * ry2; nx2 = nx2 + koh * rx2
        iy1 = jnp.maximum(cy1, by1); ix1 = jnp.maximum(cx1, bx1)
        iy2 = jnp.minimum(cy2, by2); ix2 = jnp.minimum(cx2, bx2)
        inter = jnp.maximum(iy2 - iy1, 0.0) * jnp.maximum(ix2 - ix1, 0.0)
        union = carea + barea - inter
        iou = inter / jnp.maximum(union, 1e-8)
        sc = jnp.where(iou > 0.5, _NEG_INF, sc)
        return sc, ny1, nx1, ny2, nx2

    z3 = jnp.zeros((_R3, 128), f32)
    _, ny1, nx1, ny2, nx2 = lax.fori_loop(
        0, _NMS, _nms_body, (csc0, z3, z3, z3, z3))

    narea = jnp.maximum(ny2 - ny1, 0.0) * jnp.maximum(nx2 - nx1, 0.0)
    max_iou = jnp.full((_R3, 128), -1e30, f32)
    amax = jnp.zeros((_R3, 128), f32)
    for g in range(n_gt):
        gy1 = gtb_ref[0, g, 0]; gx1 = gtb_ref[0, g, 1]
        gy2 = gtb_ref[0, g, 2]; gx2 = gtb_ref[0, g, 3]
        garea = (jnp.maximum(gy2 - gy1, 0.0) * jnp.maximum(gx2 - gx1, 0.0))
        iy1 = jnp.maximum(ny1, gy1); ix1 = jnp.maximum(nx1, gx1)
        iy2 = jnp.minimum(ny2, gy2); ix2 = jnp.minimum(nx2, gx2)
        inter = jnp.maximum(iy2 - iy1, 0.0) * jnp.maximum(ix2 - ix1, 0.0)
        iou_g = inter / jnp.maximum(narea + garea - inter, 1e-8)
        upd = iou_g > max_iou
        amax = jnp.where(upd, float(g), amax)
        max_iou = jnp.where(upd, iou_g, max_iou)

    valid3 = flat3 < float(_NMS)
    posc = jnp.where(valid3, max_iou, -1e30)
    negc = jnp.where(valid3, -max_iou, -1e30)

    lane = lax.broadcasted_iota(jnp.int32, (1, 128), 1).astype(f32)

    def _sel_body(k, carry):
        posc, negc, ry1, rx1, ry2, rx2, gx = carry
        kf = k.astype(f32)
        use_pos = k < _P
        cur = jnp.where(use_pos, posc, negc)
        m = jnp.max(cur)
        idx = jnp.min(jnp.where(cur == m, flat3, 1e9))
        oh = (flat3 == idx)
        ohf = jnp.where(oh, 1.0, 0.0)
        posc = jnp.where(oh & use_pos, -1e30, posc)
        negc = jnp.where(oh & jnp.logical_not(use_pos), -1e30, negc)
        by1 = jnp.sum(ohf * ny1); bx1 = jnp.sum(ohf * nx1)
        by2 = jnp.sum(ohf * ny2); bx2 = jnp.sum(ohf * nx2)
        bg = jnp.sum(ohf * amax)
        koh = jnp.where(lane == kf, 1.0, 0.0)
        ry1 = ry1 + koh * by1; rx1 = rx1 + koh * bx1
        ry2 = ry2 + koh * by2; rx2 = rx2 + koh * bx2
        gx = gx + koh * bg
        return posc, negc, ry1, rx1, ry2, rx2, gx

    zl = jnp.zeros((1, 128), f32)
    _, _, ry1, rx1, ry2, rx2, gx = lax.fori_loop(
        0, _T, _sel_body, (posc, negc, zl, zl, zl, zl, zl))

    ispos = lane < float(_P)
    gty1 = zl; gtx1 = zl; gty2 = zl; gtx2 = zl
    labv = jnp.full((1, 128), float(_L - 1), f32)
    for g in range(n_gt):
        selg = ispos & (gx == float(g))
        gty1 = jnp.where(selg, gtb_ref[0, g, 0], gty1)
        gtx1 = jnp.where(selg, gtb_ref[0, g, 1], gtx1)
        gty2 = jnp.where(selg, gtb_ref[0, g, 2], gty2)
        gtx2 = jnp.where(selg, gtb_ref[0, g, 3], gtx2)
        labv = jnp.where(selg, gtl_ref[0, 0, g].astype(f32), labv)

    bw = rx2 - rx1
    bh = ry2 - ry1
    bcx = rx1 + 0.5 * bw
    bcy = ry1 + 0.5 * bh
    gw = gtx2 - gtx1
    gh = gty2 - gty1
    gcx = gtx1 + 0.5 * gw
    gcy = gty1 + 0.5 * gh
    bw_s = jnp.where(bw <= 0, 1e-3, bw)
    bh_s = jnp.where(bh <= 0, 1e-3, bh)
    gw_s = jnp.where(gw <= 0, 1.0, gw)
    gh_s = jnp.where(gh <= 0, 1.0, gh)
    dx = jnp.where(gw == 0, 0.0, (gcx - bcx) / bw_s)
    dy = jnp.where(gh == 0, 0.0, (gcy - bcy) / bh_s)
    dw = jnp.where(gw == 0, 0.0, jnp.log(gw_s / bw_s))
    dh = jnp.where(gh == 0, 0.0, jnp.log(gh_s / bh_s))

    roi_ref[0, 0] = jnp.reshape(ry1, (128,))
    roi_ref[0, 1] = jnp.reshape(rx1, (128,))
    roi_ref[0, 2] = jnp.reshape(ry2, (128,))
    roi_ref[0, 3] = jnp.reshape(rx2, (128,))

    lab_i = labv.astype(jnp.int32)
    for l in range(_L):
        ohl = (lab_i == l)
        ohlf = jnp.where(ohl, 1.0, 0.0)
        dout_ref[0, 4 * l + 0] = jnp.reshape(ohlf * dy, (128,))
        dout_ref[0, 4 * l + 1] = jnp.reshape(ohlf * dx, (128,))
        dout_ref[0, 4 * l + 2] = jnp.reshape(ohlf * dh, (128,))
        dout_ref[0, 4 * l + 3] = jnp.reshape(ohlf * dw, (128,))
        lout_ref[0, l] = jnp.reshape(ohl.astype(jnp.int32), (128,))


def _crop_kernel(fm_ref, roi_ref, out_ref, *, H, W):
    f32 = jnp.float32

    def body(t, _):
        by1 = roi_ref[0, t, 0]
        bx1 = roi_ref[0, t, 1]
        by2 = roi_ref[0, t, 2]
        bx2 = roi_ref[0, t, 3]
        ystep = (by2 - by1) * (float(H - 1) / float(_CH - 1))
        xstep = (bx2 - bx1) * (float(W - 1) / float(_CW - 1))
        for k in range(_CH):
            ys = by1 * float(H - 1) + float(k) * ystep
            y0 = jnp.floor(ys)
            y0i = jnp.clip(y0, 0.0, float(H - 1)).astype(jnp.int32)
            y1i = jnp.clip(y0 + 1.0, 0.0, float(H - 1)).astype(jnp.int32)
            wy = ys - y0
            vy = jnp.logical_and(ys >= 0.0, ys <= float(H - 1))
            for l in range(_CW):
                xs = bx1 * float(W - 1) + float(l) * xstep
                x0 = jnp.floor(xs)
                x0i = jnp.clip(x0, 0.0, float(W - 1)).astype(jnp.int32)
                x1i = jnp.clip(x0 + 1.0, 0.0, float(W - 1)).astype(jnp.int32)
                wx = xs - x0
                vx = jnp.logical_and(xs >= 0.0, xs <= float(W - 1))
                v00 = fm_ref[0, y0i, x0i, :]
                v01 = fm_ref[0, y0i, x1i, :]
                v10 = fm_ref[0, y1i, x0i, :]
                v11 = fm_ref[0, y1i, x1i, :]
                top = v00 * (1.0 - wx) + v01 * wx
                bot = v10 * (1.0 - wx) + v11 * wx
                o = top * (1.0 - wy) + bot * wy
                ok = jnp.logical_and(vy, vx)
                o = jnp.where(ok, o, jnp.zeros_like(o)).astype(f32)
                out_ref[0, t, k, l, :] = o
        return 0

    lax.fori_loop(0, _T, body, 0)


def kernel(feature_map, rpn_bbox_deltas, rpn_labels, anchors, gt_boxes,
           gt_labels):
    B, N = anchors.shape[0], anchors.shape[1]
    H, W, C = feature_map.shape[1], feature_map.shape[2], feature_map.shape[3]
    n_gt = gt_boxes.shape[1]
    f32 = jnp.float32

    scores = rpn_labels.reshape(B, N)
    pad = _NPAD - N
    scores_p = jnp.pad(scores, ((0, 0), (0, pad)),
                       constant_values=-1.0).reshape(B, _R, 128)
    anch_p = jnp.pad(anchors, ((0, 0), (0, pad), (0, 0)))
    anch_p = anch_p.transpose(0, 2, 1).reshape(B, 4, _R, 128)
    delt_p = jnp.pad(rpn_bbox_deltas.reshape(B, N, 4), ((0, 0), (0, pad), (0, 0)))
    delt_p = delt_p.transpose(0, 2, 1).reshape(B, 4, _R, 128)

    sel = pl.pallas_call(
        functools.partial(_sel_kernel, n_real=N, n_gt=n_gt),
        grid=(B,),
        in_specs=[
            pl.BlockSpec((1, _R, 128), lambda b: (b, 0, 0)),
            pl.BlockSpec((1, 4, _R, 128), lambda b: (b, 0, 0, 0)),
            pl.BlockSpec((1, 4, _R, 128), lambda b: (b, 0, 0, 0)),
            pl.BlockSpec((1, n_gt, 4), lambda b: (b, 0, 0),
                         memory_space=pltpu.SMEM),
            pl.BlockSpec((1, 1, n_gt), lambda b: (b, 0, 0),
                         memory_space=pltpu.SMEM),
        ],
        out_specs=[
            pl.BlockSpec((1, 4, 128), lambda b: (b, 0, 0)),
            pl.BlockSpec((1, 4 * _L, 128), lambda b: (b, 0, 0)),
            pl.BlockSpec((1, _L, 128), lambda b: (b, 0, 0)),
        ],
        out_shape=[
            jax.ShapeDtypeStruct((B, 4, 128), f32),
            jax.ShapeDtypeStruct((B, 4 * _L, 128), f32),
            jax.ShapeDtypeStruct((B, _L, 128), jnp.int32),
        ],
        scratch_shapes=[
            pltpu.VMEM((_R, 128), f32),
            pltpu.VMEM((_R, 128), f32),
            pltpu.VMEM((8, _R, 128), f32),
            pltpu.VMEM((_NC + 128, 8), f32),
            pltpu.VMEM((8, _RC, 128), f32),
        ],
    )(scores_p, anch_p, delt_p, gt_boxes, gt_labels.reshape(B, 1, n_gt))
    roi_t, dout_t, lout_t = sel

    roi_rows = roi_t.transpose(0, 2, 1)

    final = pl.pallas_call(
        functools.partial(_crop_kernel, H=H, W=W),
        grid=(B,),
        in_specs=[
            pl.BlockSpec((1, H, W, C), lambda b: (b, 0, 0, 0)),
            pl.BlockSpec((1, _T, 4), lambda b: (b, 0, 0),
                         memory_space=pltpu.SMEM),
        ],
        out_specs=pl.BlockSpec((1, _T, _CH, _CW, C),
                               lambda b: (b, 0, 0, 0, 0)),
        out_shape=jax.ShapeDtypeStruct((B, _T, _CH, _CW, C), f32),
    )(feature_map, roi_rows)

    roi_bbox_deltas_out = dout_t.transpose(0, 2, 1)
    roi_bbox_labels = lout_t.transpose(0, 2, 1)
    return (final, lax.stop_gradient(roi_bbox_deltas_out), roi_bbox_labels)


# speedup vs baseline: 1.1222x; 1.1222x over previous
import functools

import jax
import jax.numpy as jnp
from jax import lax
from jax.experimental import pallas as pl
from jax.experimental.pallas import tpu as pltpu

_L = 21
_P = 32
_NG = 96
_T = _P + _NG
_TOPK = 2000
_NMS = 300
_NEG_INF = -1e9
_CH, _CW = 7, 7

_R = 160
_NPAD = _R * 128
_R3 = 3


_NC = 2048
_RC = 16


def _sel_kernel(scores_ref, anch_ref, delt_ref, gtb_ref, gtl_ref,
                roi_ref, dout_ref, lout_ref,
                pir_ref, rowoff_ref, vals_ref, cmp_ref, cco_ref,
                *, n_real, n_gt):
    f32 = jnp.float32
    s = scores_ref[0]
    a0 = anch_ref[0, 0]; a1 = anch_ref[0, 1]
    a2 = anch_ref[0, 2]; a3 = anch_ref[0, 3]
    d0 = delt_ref[0, 0]; d1 = delt_ref[0, 1]
    d2 = delt_ref[0, 2]; d3 = delt_ref[0, 3]

    aw = a3 - a1
    ah = a2 - a0
    acx = a1 + 0.5 * aw
    acy = a0 + 0.5 * ah
    bw = jnp.exp(d3) * aw
    bh = jnp.exp(d2) * ah
    bcx = d1 * aw + acx
    bcy = d0 * ah + acy
    y1 = bcy - 0.5 * bh
    x1 = bcx - 0.5 * bw
    y2 = bh + y1
    x2 = bw + x1
    y1 = jnp.clip(y1, 0.0, 1.0); x1 = jnp.clip(x1, 0.0, 1.0)
    y2 = jnp.clip(y2, 0.0, 1.0); x2 = jnp.clip(x2, 0.0, 1.0)
    area = jnp.maximum(y2 - y1, 0.0) * jnp.maximum(x2 - x1, 0.0)

    bits = lax.bitcast_convert_type(s, jnp.int32)

    def _bs_body(_, lohi):
        lo, hi = lohi
        mid = lo + lax.div(hi - lo, jnp.int32(2))
        cnt = jnp.sum(jnp.where(bits >= mid, 1.0, 0.0))
        ok = cnt >= float(_TOPK)
        return (jnp.where(ok, mid, lo), jnp.where(ok, hi, mid))

    lo0 = jnp.int32(0)
    hi0 = jnp.int32(2139095041)
    tau, _ = lax.fori_loop(0, 31, _bs_body, (lo0, hi0))

    cnt_gt = jnp.sum(jnp.where(bits > tau, 1.0, 0.0))
    m_needed = float(_TOPK) - cnt_gt
    tie = (bits == tau)
    tie_f = jnp.where(tie, 1.0, 0.0)
    col_i = lax.broadcasted_iota(jnp.int32, (128, 128), 0)
    col_j = lax.broadcasted_iota(jnp.int32, (128, 128), 1)
    slt_c = jnp.where(col_i < col_j, 1.0, 0.0).astype(f32)
    row_i = lax.broadcasted_iota(jnp.int32, (_R, _R), 0)
    row_j = lax.broadcasted_iota(jnp.int32, (_R, _R), 1)
    slt_r = jnp.where(row_j < row_i, 1.0, 0.0).astype(f32)
    within = lax.dot_general(tie_f, slt_c, (((1,), (0,)), ((), ())),
                             preferred_element_type=f32)
    rowsum = jnp.sum(tie_f, axis=1, keepdims=True)
    before = lax.dot_general(slt_r, rowsum, (((1,), (0,)), ((), ())),
                             preferred_element_type=f32)
    prefix = within + before
    eligible = (bits > tau) | (tie & (prefix < m_needed))

    flat = (lax.broadcasted_iota(jnp.int32, (_R, 128), 0) * 128
            + lax.broadcasted_iota(jnp.int32, (_R, 128), 1)).astype(f32)
    flat3 = (lax.broadcasted_iota(jnp.int32, (_R3, 128), 0) * 128
             + lax.broadcasted_iota(jnp.int32, (_R3, 128), 1)).astype(f32)

    elig_f = jnp.where(eligible, 1.0, 0.0)
    within_e = lax.dot_general(elig_f, slt_c, (((1,), (0,)), ((), ())),
                               preferred_element_type=f32)
    rowsum_e = jnp.sum(elig_f, axis=1, keepdims=True)
    before_e = lax.dot_general(slt_r, rowsum_e, (((1,), (0,)), ((), ())),
                               preferred_element_type=f32)

    pir_ref[...] = jnp.where(eligible, within_e, 1e9)
    rowoff_ref[...] = jnp.broadcast_to(before_e, (_R, 128))
    vals_ref[0] = y1; vals_ref[1] = x1; vals_ref[2] = y2; vals_ref[3] = x2
    vals_ref[4] = area; vals_ref[5] = s; vals_ref[6] = flat
    vals_ref[7] = jnp.zeros((_R, 128), f32)

    tgt256 = lax.broadcasted_iota(jnp.int32, (256, 128), 0).astype(f32)

    cmp_ref[...] = jnp.zeros((_NC + 128, 8), f32)

    def _comp_body(r, _):
        pir = pir_ref[pl.ds(r, 1), :]
        p = rowoff_ref[pl.ds(r, 1), :][0, 0].astype(jnp.int32)
        tr0 = lax.div(p, jnp.int32(128))
        off = (p - tr0 * 128).astype(f32)
        tloc = pir + off
        T = jnp.where(tgt256 == tloc, 1.0, 0.0)
        vrow = vals_ref[:, pl.ds(r, 1), :].reshape(8, 128)
        V = vrow.T
        C = lax.dot_general(T, V, (((1,), (0,)), ((), ())),
                            preferred_element_type=f32,
                            precision=lax.Precision.HIGHEST)
        cmp_ref[pl.ds(tr0 * 128, 256), :] += C
        return 0

    lax.fori_loop(0, _R, _comp_body, 0)

    for b in range(_RC):
        blk = cmp_ref[pl.ds(b * 128, 128), :]
        cco_ref[:, b, :] = blk.T

    cposf = (lax.broadcasted_iota(jnp.int32, (_RC, 128), 0) * 128
             + lax.broadcasted_iota(jnp.int32, (_RC, 128), 1)).astype(f32)
    valid_c = cposf < float(_TOPK)
    cy1 = cco_ref[0]; cx1 = cco_ref[1]; cy2 = cco_ref[2]; cx2 = cco_ref[3]
    carea = cco_ref[4]
    csc0 = jnp.where(valid_c, cco_ref[5], _NEG_INF)
    cflat = jnp.where(valid_c, cco_ref[6], 2e9 + cposf)

    def _nms_body(k, carry):
        sc, ny1, nx1, ny2, nx2 = carry
        m = jnp.max(sc, keepdims=True)
        idx = jnp.min(jnp.where(sc == m, cflat, 4e9), keepdims=True)
        ohf = jnp.where(cflat == idx, 1.0, 0.0)
        by1 = jnp.sum(ohf * cy1, keepdims=True)
        bx1 = jnp.sum(ohf * cx1, keepdims=True)
        by2 = jnp.sum(ohf * cy2, keepdims=True)
        bx2 = jnp.sum(ohf * cx2, keepdims=True)
        barea = jnp.sum(ohf * carea, keepdims=True)
        valid = m > _NEG_INF * 0.5
        ry1 = jnp.where(valid, by1, 0.0); rx1 = jnp.where(valid, bx1, 0.0)
        ry2 = jnp.where(valid, by2, 0.0); rx2 = jnp.where(valid, bx2, 0.0)
        koh = jnp.where(flat3 == k.astype(f32), 1.0, 0.0)
        ny1 = ny1 + koh * ry1; nx1 = nx1 + koh * rx1
        ny2 = ny2 + koh * ry2; nx2 = nx2 + koh * rx2
        iy1 = jnp.maximum(cy1, by1); ix1 = jnp.maximum(cx1, bx1)
        iy2 = jnp.minimum(cy2, by2); ix2 = jnp.minimum(cx2, bx2)
        inter = jnp.maximum(iy2 - iy1, 0.0) * jnp.maximum(ix2 - ix1, 0.0)
        union = carea + barea - inter
        iou = inter / jnp.maximum(union, 1e-8)
        sc = jnp.where(iou > 0.5, _NEG_INF, sc)
        return sc, ny1, nx1, ny2, nx2

    z3 = jnp.zeros((_R3, 128), f32)
    _, ny1, nx1, ny2, nx2 = lax.fori_loop(
        0, _NMS, _nms_body, (csc0, z3, z3, z3, z3))

    narea = jnp.maximum(ny2 - ny1, 0.0) * jnp.maximum(nx2 - nx1, 0.0)
    max_iou = jnp.full((_R3, 128), -1e30, f32)
    amax = jnp.zeros((_R3, 128), f32)
    for g in range(n_gt):
        gy1 = gtb_ref[0, g, 0]; gx1 = gtb_ref[0, g, 1]
        gy2 = gtb_ref[0, g, 2]; gx2 = gtb_ref[0, g, 3]
        garea = (jnp.maximum(gy2 - gy1, 0.0) * jnp.maximum(gx2 - gx1, 0.0))
        iy1 = jnp.maximum(ny1, gy1); ix1 = jnp.maximum(nx1, gx1)
        iy2 = jnp.minimum(ny2, gy2); ix2 = jnp.minimum(nx2, gx2)
        inter = jnp.maximum(iy2 - iy1, 0.0) * jnp.maximum(ix2 - ix1, 0.0)
        iou_g = inter / jnp.maximum(narea + garea - inter, 1e-8)
        upd = iou_g > max_iou
        amax = jnp.where(upd, float(g), amax)
        max_iou = jnp.where(upd, iou_g, max_iou)

    valid3 = flat3 < float(_NMS)
    posc = jnp.where(valid3, max_iou, -1e30)
    negc = jnp.where(valid3, -max_iou, -1e30)

    lane = lax.broadcasted_iota(jnp.int32, (1, 128), 1).astype(f32)

    def _sel_body(k, carry):
        posc, negc, ry1, rx1, ry2, rx2, gx = carry
        kf = k.astype(f32)
        use_pos = k < _P
        cur = jnp.where(use_pos, posc, negc)
        m = jnp.max(cur, keepdims=True)
        idx = jnp.min(jnp.where(cur == m, flat3, 1e9), keepdims=True)
        oh = (flat3 == idx)
        ohf = jnp.where(oh, 1.0, 0.0)
        posc = jnp.where(oh & use_pos, -1e30, posc)
        negc = jnp.where(oh & jnp.logical_not(use_pos), -1e30, negc)
        by1 = jnp.sum(ohf * ny1, keepdims=True)
        bx1 = jnp.sum(ohf * nx1, keepdims=True)
        by2 = jnp.sum(ohf * ny2, keepdims=True)
        bx2 = jnp.sum(ohf * nx2, keepdims=True)
        bg = jnp.sum(ohf * amax, keepdims=True)
        koh = jnp.where(lane == kf, 1.0, 0.0)
        ry1 = ry1 + koh * by1; rx1 = rx1 + koh * bx1
        ry2 = ry2 + koh * by2; rx2 = rx2 + koh * bx2
        gx = gx + koh * bg
        return posc, negc, ry1, rx1, ry2, rx2, gx

    zl = jnp.zeros((1, 128), f32)
    _, _, ry1, rx1, ry2, rx2, gx = lax.fori_loop(
        0, _T, _sel_body, (posc, negc, zl, zl, zl, zl, zl))

    ispos = lane < float(_P)
    gty1 = zl; gtx1 = zl; gty2 = zl; gtx2 = zl
    labv = jnp.full((1, 128), float(_L - 1), f32)
    for g in range(n_gt):
        selg = ispos & (gx == float(g))
        gty1 = jnp.where(selg, gtb_ref[0, g, 0], gty1)
        gtx1 = jnp.where(selg, gtb_ref[0, g, 1], gtx1)
        gty2 = jnp.where(selg, gtb_ref[0, g, 2], gty2)
        gtx2 = jnp.where(selg, gtb_ref[0, g, 3], gtx2)
        labv = jnp.where(selg, gtl_ref[0, 0, g].astype(f32), labv)

    bw = rx2 - rx1
    bh = ry2 - ry1
    bcx = rx1 + 0.5 * bw
    bcy = ry1 + 0.5 * bh
    gw = gtx2 - gtx1
    gh = gty2 - gty1
    gcx = gtx1 + 0.5 * gw
    gcy = gty1 + 0.5 * gh
    bw_s = jnp.where(bw <= 0, 1e-3, bw)
    bh_s = jnp.where(bh <= 0, 1e-3, bh)
    gw_s = jnp.where(gw <= 0, 1.0, gw)
    gh_s = jnp.where(gh <= 0, 1.0, gh)
    dx = jnp.where(gw == 0, 0.0, (gcx - bcx) / bw_s)
    dy = jnp.where(gh == 0, 0.0, (gcy - bcy) / bh_s)
    dw = jnp.where(gw == 0, 0.0, jnp.log(gw_s / bw_s))
    dh = jnp.where(gh == 0, 0.0, jnp.log(gh_s / bh_s))

    roi_ref[0, 0] = jnp.reshape(ry1, (128,))
    roi_ref[0, 1] = jnp.reshape(rx1, (128,))
    roi_ref[0, 2] = jnp.reshape(ry2, (128,))
    roi_ref[0, 3] = jnp.reshape(rx2, (128,))

    lab_i = labv.astype(jnp.int32)
    for l in range(_L):
        ohl = (lab_i == l)
        ohlf = jnp.where(ohl, 1.0, 0.0)
        dout_ref[0, 4 * l + 0] = jnp.reshape(ohlf * dy, (128,))
        dout_ref[0, 4 * l + 1] = jnp.reshape(ohlf * dx, (128,))
        dout_ref[0, 4 * l + 2] = jnp.reshape(ohlf * dh, (128,))
        dout_ref[0, 4 * l + 3] = jnp.reshape(ohlf * dw, (128,))
        lout_ref[0, l] = jnp.reshape(ohl.astype(jnp.int32), (128,))


def _crop_kernel(fm_ref, roi_ref, out_ref, *, H, W):
    f32 = jnp.float32

    def body(t, _):
        by1 = roi_ref[0, t, 0]
        bx1 = roi_ref[0, t, 1]
        by2 = roi_ref[0, t, 2]
        bx2 = roi_ref[0, t, 3]
        ystep = (by2 - by1) * (float(H - 1) / float(_CH - 1))
        xstep = (bx2 - bx1) * (float(W - 1) / float(_CW - 1))
        for k in range(_CH):
            ys = by1 * float(H - 1) + float(k) * ystep
            y0 = jnp.floor(ys)
            y0i = jnp.clip(y0, 0.0, float(H - 1)).astype(jnp.int32)
            y1i = jnp.clip(y0 + 1.0, 0.0, float(H - 1)).astype(jnp.int32)
            wy = ys - y0
            vy = jnp.logical_and(ys >= 0.0, ys <= float(H - 1))
            for l in range(_CW):
                xs = bx1 * float(W - 1) + float(l) * xstep
                x0 = jnp.floor(xs)
                x0i = jnp.clip(x0, 0.0, float(W - 1)).astype(jnp.int32)
                x1i = jnp.clip(x0 + 1.0, 0.0, float(W - 1)).astype(jnp.int32)
                wx = xs - x0
                vx = jnp.logical_and(xs >= 0.0, xs <= float(W - 1))
                v00 = fm_ref[0, y0i, x0i, :]
                v01 = fm_ref[0, y0i, x1i, :]
                v10 = fm_ref[0, y1i, x0i, :]
                v11 = fm_ref[0, y1i, x1i, :]
                top = v00 * (1.0 - wx) + v01 * wx
                bot = v10 * (1.0 - wx) + v11 * wx
                o = top * (1.0 - wy) + bot * wy
                ok = jnp.logical_and(vy, vx)
                o = jnp.where(ok, o, jnp.zeros_like(o)).astype(f32)
                out_ref[0, t, k, l, :] = o
        return 0

    lax.fori_loop(0, _T, body, 0)


def kernel(feature_map, rpn_bbox_deltas, rpn_labels, anchors, gt_boxes,
           gt_labels):
    B, N = anchors.shape[0], anchors.shape[1]
    H, W, C = feature_map.shape[1], feature_map.shape[2], feature_map.shape[3]
    n_gt = gt_boxes.shape[1]
    f32 = jnp.float32

    scores = rpn_labels.reshape(B, N)
    pad = _NPAD - N
    scores_p = jnp.pad(scores, ((0, 0), (0, pad)),
                       constant_values=-1.0).reshape(B, _R, 128)
    anch_p = jnp.pad(anchors, ((0, 0), (0, pad), (0, 0)))
    anch_p = anch_p.transpose(0, 2, 1).reshape(B, 4, _R, 128)
    delt_p = jnp.pad(rpn_bbox_deltas.reshape(B, N, 4), ((0, 0), (0, pad), (0, 0)))
    delt_p = delt_p.transpose(0, 2, 1).reshape(B, 4, _R, 128)

    sel = pl.pallas_call(
        functools.partial(_sel_kernel, n_real=N, n_gt=n_gt),
        grid=(B,),
        in_specs=[
            pl.BlockSpec((1, _R, 128), lambda b: (b, 0, 0)),
            pl.BlockSpec((1, 4, _R, 128), lambda b: (b, 0, 0, 0)),
            pl.BlockSpec((1, 4, _R, 128), lambda b: (b, 0, 0, 0)),
            pl.BlockSpec((1, n_gt, 4), lambda b: (b, 0, 0),
                         memory_space=pltpu.SMEM),
            pl.BlockSpec((1, 1, n_gt), lambda b: (b, 0, 0),
                         memory_space=pltpu.SMEM),
        ],
        out_specs=[
            pl.BlockSpec((1, 4, 128), lambda b: (b, 0, 0)),
            pl.BlockSpec((1, 4 * _L, 128), lambda b: (b, 0, 0)),
            pl.BlockSpec((1, _L, 128), lambda b: (b, 0, 0)),
        ],
        out_shape=[
            jax.ShapeDtypeStruct((B, 4, 128), f32),
            jax.ShapeDtypeStruct((B, 4 * _L, 128), f32),
            jax.ShapeDtypeStruct((B, _L, 128), jnp.int32),
        ],
        scratch_shapes=[
            pltpu.VMEM((_R, 128), f32),
            pltpu.VMEM((_R, 128), f32),
            pltpu.VMEM((8, _R, 128), f32),
            pltpu.VMEM((_NC + 128, 8), f32),
            pltpu.VMEM((8, _RC, 128), f32),
        ],
    )(scores_p, anch_p, delt_p, gt_boxes, gt_labels.reshape(B, 1, n_gt))
    roi_t, dout_t, lout_t = sel

    roi_rows = roi_t.transpose(0, 2, 1)

    final = pl.pallas_call(
        functools.partial(_crop_kernel, H=H, W=W),
        grid=(B,),
        in_specs=[
            pl.BlockSpec((1, H, W, C), lambda b: (b, 0, 0, 0)),
            pl.BlockSpec((1, _T, 4), lambda b: (b, 0, 0),
                         memory_space=pltpu.SMEM),
        ],
        out_specs=pl.BlockSpec((1, _T, _CH, _CW, C),
                               lambda b: (b, 0, 0, 0, 0)),
        out_shape=jax.ShapeDtypeStruct((B, _T, _CH, _CW, C), f32),
    )(feature_map, roi_rows)

    roi_bbox_deltas_out = dout_t.transpose(0, 2, 1)
    roi_bbox_labels = lout_t.transpose(0, 2, 1)
    return (final, lax.stop_gradient(roi_bbox_deltas_out), roi_bbox_labels)


# speedup vs baseline: 1.2334x; 1.0991x over previous
import functools

import jax
import jax.numpy as jnp
from jax import lax
from jax.experimental import pallas as pl
from jax.experimental.pallas import tpu as pltpu

_L = 21
_P = 32
_NG = 96
_T = _P + _NG
_TOPK = 2000
_NMS = 300
_NEG_INF = -1e9
_CH, _CW = 7, 7

_R = 160
_NPAD = _R * 128
_R3 = 3


def _sel_kernel(scores_ref, anch_ref, delt_ref, gtb_ref, gtl_ref,
                roi_ref, dout_ref, lout_ref, *, n_real, n_gt):
    f32 = jnp.float32
    s = scores_ref[0]
    a0 = anch_ref[0, 0]; a1 = anch_ref[0, 1]
    a2 = anch_ref[0, 2]; a3 = anch_ref[0, 3]
    d0 = delt_ref[0, 0]; d1 = delt_ref[0, 1]
    d2 = delt_ref[0, 2]; d3 = delt_ref[0, 3]

    aw = a3 - a1
    ah = a2 - a0
    acx = a1 + 0.5 * aw
    acy = a0 + 0.5 * ah
    bw = jnp.exp(d3) * aw
    bh = jnp.exp(d2) * ah
    bcx = d1 * aw + acx
    bcy = d0 * ah + acy
    y1 = bcy - 0.5 * bh
    x1 = bcx - 0.5 * bw
    y2 = bh + y1
    x2 = bw + x1
    y1 = jnp.clip(y1, 0.0, 1.0); x1 = jnp.clip(x1, 0.0, 1.0)
    y2 = jnp.clip(y2, 0.0, 1.0); x2 = jnp.clip(x2, 0.0, 1.0)
    area = jnp.maximum(y2 - y1, 0.0) * jnp.maximum(x2 - x1, 0.0)

    bits = lax.bitcast_convert_type(s, jnp.int32)

    def _bs_body(_, lohi):
        lo, hi = lohi
        mid = lo + lax.div(hi - lo, jnp.int32(2))
        cnt = jnp.sum(jnp.where(bits >= mid, 1.0, 0.0))
        ok = cnt >= float(_TOPK)
        return (jnp.where(ok, mid, lo), jnp.where(ok, hi, mid))

    lo0 = jnp.int32(0)
    hi0 = jnp.int32(2139095041)
    tau, _ = lax.fori_loop(0, 31, _bs_body, (lo0, hi0))

    cnt_gt = jnp.sum(jnp.where(bits > tau, 1.0, 0.0))
    m_needed = float(_TOPK) - cnt_gt
    tie = (bits == tau)
    tie_f = jnp.where(tie, 1.0, 0.0)
    col_i = lax.broadcasted_iota(jnp.int32, (128, 128), 0)
    col_j = lax.broadcasted_iota(jnp.int32, (128, 128), 1)
    slt_c = jnp.where(col_i < col_j, 1.0, 0.0).astype(f32)
    row_i = lax.broadcasted_iota(jnp.int32, (_R, _R), 0)
    row_j = lax.broadcasted_iota(jnp.int32, (_R, _R), 1)
    slt_r = jnp.where(row_j < row_i, 1.0, 0.0).astype(f32)
    within = lax.dot_general(tie_f, slt_c, (((1,), (0,)), ((), ())),
                             preferred_element_type=f32)
    rowsum = jnp.sum(tie_f, axis=1, keepdims=True)
    before = lax.dot_general(slt_r, rowsum, (((1,), (0,)), ((), ())),
                             preferred_element_type=f32)
    prefix = within + before
    eligible = (bits > tau) | (tie & (prefix < m_needed))
    sc = jnp.where(eligible, s, _NEG_INF)

    flat = (lax.broadcasted_iota(jnp.int32, (_R, 128), 0) * 128
            + lax.broadcasted_iota(jnp.int32, (_R, 128), 1)).astype(f32)
    flat3 = (lax.broadcasted_iota(jnp.int32, (_R3, 128), 0) * 128
             + lax.broadcasted_iota(jnp.int32, (_R3, 128), 1)).astype(f32)

    def _nms_body(k, carry):
        sc, ny1, nx1, ny2, nx2 = carry
        m = jnp.max(sc)
        idx = jnp.min(jnp.where(sc == m, flat, 1e9))
        oh = (flat == idx)
        ohf = jnp.where(oh, 1.0, 0.0)
        by1 = jnp.sum(ohf * y1); bx1 = jnp.sum(ohf * x1)
        by2 = jnp.sum(ohf * y2); bx2 = jnp.sum(ohf * x2)
        barea = jnp.maximum(by2 - by1, 0.0) * jnp.maximum(bx2 - bx1, 0.0)
        valid = m > _NEG_INF * 0.5
        ry1 = jnp.where(valid, by1, 0.0); rx1 = jnp.where(valid, bx1, 0.0)
        ry2 = jnp.where(valid, by2, 0.0); rx2 = jnp.where(valid, bx2, 0.0)
        koh = jnp.where(flat3 == k.astype(f32), 1.0, 0.0)
        ny1 = ny1 + koh * ry1; nx1 = nx1 + koh * rx1
        ny2 = ny2 + koh * ry2; nx2 = nx2 + koh * rx2
        iy1 = jnp.maximum(y1, by1); ix1 = jnp.maximum(x1, bx1)
        iy2 = jnp.minimum(y2, by2); ix2 = jnp.minimum(x2, bx2)
        inter = jnp.maximum(iy2 - iy1, 0.0) * jnp.maximum(ix2 - ix1, 0.0)
        union = area + barea - inter
        iou = inter / jnp.maximum(union, 1e-8)
        sc = jnp.where(iou > 0.5, _NEG_INF, sc)
        return sc, ny1, nx1, ny2, nx2

    z3 = jnp.zeros((_R3, 128), f32)
    _, ny1, nx1, ny2, nx2 = lax.fori_loop(
        0, _NMS, _nms_body, (sc, z3, z3, z3, z3))

    narea = jnp.maximum(ny2 - ny1, 0.0) * jnp.maximum(nx2 - nx1, 0.0)
    max_iou = jnp.full((_R3, 128), -1e30, f32)
    amax = jnp.zeros((_R3, 128), f32)
    for g in range(n_gt):
        gy1 = gtb_ref[0, g, 0]; gx1 = gtb_ref[0, g, 1]
        gy2 = gtb_ref[0, g, 2]; gx2 = gtb_ref[0, g, 3]
        garea = (jnp.maximum(gy2 - gy1, 0.0) * jnp.maximum(gx2 - gx1, 0.0))
        iy1 = jnp.maximum(ny1, gy1); ix1 = jnp.maximum(nx1, gx1)
        iy2 = jnp.minimum(ny2, gy2); ix2 = jnp.minimum(nx2, gx2)
        inter = jnp.maximum(iy2 - iy1, 0.0) * jnp.maximum(ix2 - ix1, 0.0)
        iou_g = inter / jnp.maximum(narea + garea - inter, 1e-8)
        upd = iou_g > max_iou
        amax = jnp.where(upd, float(g), amax)
        max_iou = jnp.where(upd, iou_g, max_iou)

    valid3 = flat3 < float(_NMS)
    posc = jnp.where(valid3, max_iou, -1e30)
    negc = jnp.where(valid3, -max_iou, -1e30)

    lane = lax.broadcasted_iota(jnp.int32, (1, 128), 1).astype(f32)

    def _sel_body(k, carry):
        posc, negc, ry1, rx1, ry2, rx2, gx = carry
        kf = k.astype(f32)
        use_pos = k < _P
        cur = jnp.where(use_pos, posc, negc)
        m = jnp.max(cur)
        idx = jnp.min(jnp.where(cur == m, flat3, 1e9))
        oh = (flat3 == idx)
        ohf = jnp.where(oh, 1.0, 0.0)
        posc = jnp.where(oh & use_pos, -1e30, posc)
        negc = jnp.where(oh & jnp.logical_not(use_pos), -1e30, negc)
        by1 = jnp.sum(ohf * ny1); bx1 = jnp.sum(ohf * nx1)
        by2 = jnp.sum(ohf * ny2); bx2 = jnp.sum(ohf * nx2)
        bg = jnp.sum(ohf * amax)
        koh = jnp.where(lane == kf, 1.0, 0.0)
        ry1 = ry1 + koh * by1; rx1 = rx1 + koh * bx1
        ry2 = ry2 + koh * by2; rx2 = rx2 + koh * bx2
        gx = gx + koh * bg
        return posc, negc, ry1, rx1, ry2, rx2, gx

    zl = jnp.zeros((1, 128), f32)
    _, _, ry1, rx1, ry2, rx2, gx = lax.fori_loop(
        0, _T, _sel_body, (posc, negc, zl, zl, zl, zl, zl))

    ispos = lane < float(_P)
    gty1 = zl; gtx1 = zl; gty2 = zl; gtx2 = zl
    labv = jnp.full((1, 128), float(_L - 1), f32)
    for g in range(n_gt):
        selg = ispos & (gx == float(g))
        gty1 = jnp.where(selg, gtb_ref[0, g, 0], gty1)
        gtx1 = jnp.where(selg, gtb_ref[0, g, 1], gtx1)
        gty2 = jnp.where(selg, gtb_ref[0, g, 2], gty2)
        gtx2 = jnp.where(selg, gtb_ref[0, g, 3], gtx2)
        labv = jnp.where(selg, gtl_ref[0, 0, g].astype(f32), labv)

    bw = rx2 - rx1
    bh = ry2 - ry1
    bcx = rx1 + 0.5 * bw
    bcy = ry1 + 0.5 * bh
    gw = gtx2 - gtx1
    gh = gty2 - gty1
    gcx = gtx1 + 0.5 * gw
    gcy = gty1 + 0.5 * gh
    bw_s = jnp.where(bw <= 0, 1e-3, bw)
    bh_s = jnp.where(bh <= 0, 1e-3, bh)
    gw_s = jnp.where(gw <= 0, 1.0, gw)
    gh_s = jnp.where(gh <= 0, 1.0, gh)
    dx = jnp.where(gw == 0, 0.0, (gcx - bcx) / bw_s)
    dy = jnp.where(gh == 0, 0.0, (gcy - bcy) / bh_s)
    dw = jnp.where(gw == 0, 0.0, jnp.log(gw_s / bw_s))
    dh = jnp.where(gh == 0, 0.0, jnp.log(gh_s / bh_s))

    roi_ref[0, 0] = jnp.reshape(ry1, (128,))
    roi_ref[0, 1] = jnp.reshape(rx1, (128,))
    roi_ref[0, 2] = jnp.reshape(ry2, (128,))
    roi_ref[0, 3] = jnp.reshape(rx2, (128,))

    lab_i = labv.astype(jnp.int32)
    for l in range(_L):
        ohl = (lab_i == l)
        ohlf = jnp.where(ohl, 1.0, 0.0)
        dout_ref[0, 4 * l + 0] = jnp.reshape(ohlf * dy, (128,))
        dout_ref[0, 4 * l + 1] = jnp.reshape(ohlf * dx, (128,))
        dout_ref[0, 4 * l + 2] = jnp.reshape(ohlf * dh, (128,))
        dout_ref[0, 4 * l + 3] = jnp.reshape(ohlf * dw, (128,))
        lout_ref[0, l] = jnp.reshape(ohl.astype(jnp.int32), (128,))


def _crop_kernel(fm_ref, roi_ref, out_ref, *, H, W):
    f32 = jnp.float32

    def body(t, _):
        by1 = roi_ref[0, t, 0]
        bx1 = roi_ref[0, t, 1]
        by2 = roi_ref[0, t, 2]
        bx2 = roi_ref[0, t, 3]
        ystep = (by2 - by1) * (float(H - 1) / float(_CH - 1))
        xstep = (bx2 - bx1) * (float(W - 1) / float(_CW - 1))
        for k in range(_CH):
            ys = by1 * float(H - 1) + float(k) * ystep
            y0 = jnp.floor(ys)
            y0i = jnp.clip(y0, 0.0, float(H - 1)).astype(jnp.int32)
            y1i = jnp.clip(y0 + 1.0, 0.0, float(H - 1)).astype(jnp.int32)
            wy = ys - y0
            vy = jnp.logical_and(ys >= 0.0, ys <= float(H - 1))
            for l in range(_CW):
                xs = bx1 * float(W - 1) + float(l) * xstep
                x0 = jnp.floor(xs)
                x0i = jnp.clip(x0, 0.0, float(W - 1)).astype(jnp.int32)
                x1i = jnp.clip(x0 + 1.0, 0.0, float(W - 1)).astype(jnp.int32)
                wx = xs - x0
                vx = jnp.logical_and(xs >= 0.0, xs <= float(W - 1))
                v00 = fm_ref[0, y0i, x0i, :]
                v01 = fm_ref[0, y0i, x1i, :]
                v10 = fm_ref[0, y1i, x0i, :]
                v11 = fm_ref[0, y1i, x1i, :]
                top = v00 * (1.0 - wx) + v01 * wx
                bot = v10 * (1.0 - wx) + v11 * wx
                o = top * (1.0 - wy) + bot * wy
                ok = jnp.logical_and(vy, vx)
                o = jnp.where(ok, o, jnp.zeros_like(o)).astype(f32)
                out_ref[0, t, k, l, :] = o
        return 0

    lax.fori_loop(0, _T, body, 0)


def kernel(feature_map, rpn_bbox_deltas, rpn_labels, anchors, gt_boxes,
           gt_labels):
    B, N = anchors.shape[0], anchors.shape[1]
    H, W, C = feature_map.shape[1], feature_map.shape[2], feature_map.shape[3]
    n_gt = gt_boxes.shape[1]
    f32 = jnp.float32

    scores = rpn_labels.reshape(B, N)
    pad = _NPAD - N
    scores_p = jnp.pad(scores, ((0, 0), (0, pad)),
                       constant_values=-1.0).reshape(B, _R, 128)
    anch_p = jnp.pad(anchors, ((0, 0), (0, pad), (0, 0)))
    anch_p = anch_p.transpose(0, 2, 1).reshape(B, 4, _R, 128)
    delt_p = jnp.pad(rpn_bbox_deltas.reshape(B, N, 4), ((0, 0), (0, pad), (0, 0)))
    delt_p = delt_p.transpose(0, 2, 1).reshape(B, 4, _R, 128)

    sel = pl.pallas_call(
        functools.partial(_sel_kernel, n_real=N, n_gt=n_gt),
        grid=(B,),
        in_specs=[
            pl.BlockSpec((1, _R, 128), lambda b: (b, 0, 0)),
            pl.BlockSpec((1, 4, _R, 128), lambda b: (b, 0, 0, 0)),
            pl.BlockSpec((1, 4, _R, 128), lambda b: (b, 0, 0, 0)),
            pl.BlockSpec((1, n_gt, 4), lambda b: (b, 0, 0),
                         memory_space=pltpu.SMEM),
            pl.BlockSpec((1, 1, n_gt), lambda b: (b, 0, 0),
                         memory_space=pltpu.SMEM),
        ],
        out_specs=[
            pl.BlockSpec((1, 4, 128), lambda b: (b, 0, 0)),
            pl.BlockSpec((1, 4 * _L, 128), lambda b: (b, 0, 0)),
            pl.BlockSpec((1, _L, 128), lambda b: (b, 0, 0)),
        ],
        out_shape=[
            jax.ShapeDtypeStruct((B, 4, 128), f32),
            jax.ShapeDtypeStruct((B, 4 * _L, 128), f32),
            jax.ShapeDtypeStruct((B, _L, 128), jnp.int32),
        ],
    )(scores_p, anch_p, delt_p, gt_boxes, gt_labels.reshape(B, 1, n_gt))
    roi_t, dout_t, lout_t = sel

    roi_rows = roi_t.transpose(0, 2, 1)

    final = pl.pallas_call(
        functools.partial(_crop_kernel, H=H, W=W),
        grid=(B,),
        in_specs=[
            pl.BlockSpec((1, H, W, C), lambda b: (b, 0, 0, 0)),
            pl.BlockSpec((1, _T, 4), lambda b: (b, 0, 0),
                         memory_space=pltpu.SMEM),
        ],
        out_specs=pl.BlockSpec((1, _T, _CH, _CW, C),
                               lambda b: (b, 0, 0, 0, 0)),
        out_shape=jax.ShapeDtypeStruct((B, _T, _CH, _CW, C), f32),
    )(feature_map, roi_rows)

    roi_bbox_deltas_out = dout_t.transpose(0, 2, 1)
    roi_bbox_labels = lout_t.transpose(0, 2, 1)
    return (final, lax.stop_gradient(roi_bbox_deltas_out), roi_bbox_labels)


# speedup vs baseline: 1.2524x; 1.0154x over previous
import functools

import jax
import jax.numpy as jnp
from jax import lax
from jax.experimental import pallas as pl
from jax.experimental.pallas import tpu as pltpu

_L = 21
_P = 32
_NG = 96
_T = _P + _NG
_TOPK = 2000
_NMS = 300
_NEG_INF = -1e9
_CH, _CW = 7, 7

_R = 160
_NPAD = _R * 128
_R3 = 3


def _sel_kernel(scores_ref, anch_ref, delt_ref, gtb_ref, gtl_ref,
                roi_ref, dout_ref, lout_ref, *, n_real, n_gt):
    f32 = jnp.float32
    s = scores_ref[0]
    a0 = anch_ref[0, 0]; a1 = anch_ref[0, 1]
    a2 = anch_ref[0, 2]; a3 = anch_ref[0, 3]
    d0 = delt_ref[0, 0]; d1 = delt_ref[0, 1]
    d2 = delt_ref[0, 2]; d3 = delt_ref[0, 3]

    aw = a3 - a1
    ah = a2 - a0
    acx = a1 + 0.5 * aw
    acy = a0 + 0.5 * ah
    bw = jnp.exp(d3) * aw
    bh = jnp.exp(d2) * ah
    bcx = d1 * aw + acx
    bcy = d0 * ah + acy
    y1 = bcy - 0.5 * bh
    x1 = bcx - 0.5 * bw
    y2 = bh + y1
    x2 = bw + x1
    y1 = jnp.clip(y1, 0.0, 1.0); x1 = jnp.clip(x1, 0.0, 1.0)
    y2 = jnp.clip(y2, 0.0, 1.0); x2 = jnp.clip(x2, 0.0, 1.0)
    area = jnp.maximum(y2 - y1, 0.0) * jnp.maximum(x2 - x1, 0.0)

    bits = lax.bitcast_convert_type(s, jnp.int32)

    def _bs_body(_, lohi):
        lo, hi = lohi
        mid = lo + lax.div(hi - lo, jnp.int32(2))
        cnt = jnp.sum(jnp.where(bits >= mid, 1.0, 0.0))
        ok = cnt >= float(_TOPK)
        return (jnp.where(ok, mid, lo), jnp.where(ok, hi, mid))

    lo0 = jnp.int32(0)
    hi0 = jnp.int32(2139095041)
    tau, _ = lax.fori_loop(0, 31, _bs_body, (lo0, hi0))

    cnt_gt = jnp.sum(jnp.where(bits > tau, 1.0, 0.0))
    m_needed = float(_TOPK) - cnt_gt
    tie = (bits == tau)
    tie_f = jnp.where(tie, 1.0, 0.0)
    col_i = lax.broadcasted_iota(jnp.int32, (128, 128), 0)
    col_j = lax.broadcasted_iota(jnp.int32, (128, 128), 1)
    slt_c = jnp.where(col_i < col_j, 1.0, 0.0).astype(f32)
    row_i = lax.broadcasted_iota(jnp.int32, (_R, _R), 0)
    row_j = lax.broadcasted_iota(jnp.int32, (_R, _R), 1)
    slt_r = jnp.where(row_j < row_i, 1.0, 0.0).astype(f32)
    within = lax.dot_general(tie_f, slt_c, (((1,), (0,)), ((), ())),
                             preferred_element_type=f32)
    rowsum = jnp.sum(tie_f, axis=1, keepdims=True)
    before = lax.dot_general(slt_r, rowsum, (((1,), (0,)), ((), ())),
                             preferred_element_type=f32)
    prefix = within + before
    eligible = (bits > tau) | (tie & (prefix < m_needed))
    sc = jnp.where(eligible, s, _NEG_INF)

    flat = (lax.broadcasted_iota(jnp.int32, (_R, 128), 0) * 128
            + lax.broadcasted_iota(jnp.int32, (_R, 128), 1)).astype(f32)
    flat3 = (lax.broadcasted_iota(jnp.int32, (_R3, 128), 0) * 128
             + lax.broadcasted_iota(jnp.int32, (_R3, 128), 1)).astype(f32)

    def _nms_body(k, carry):
        sc, ny1, nx1, ny2, nx2 = carry
        m = jnp.max(sc)
        idx = jnp.min(jnp.where(sc == m, flat, 1e9))
        oh = (flat == idx)
        ohf = jnp.where(oh, 1.0, 0.0)
        by1 = jnp.sum(ohf * y1); bx1 = jnp.sum(ohf * x1)
        by2 = jnp.sum(ohf * y2); bx2 = jnp.sum(ohf * x2)
        barea = jnp.maximum(by2 - by1, 0.0) * jnp.maximum(bx2 - bx1, 0.0)
        valid = m > _NEG_INF * 0.5
        ry1 = jnp.where(valid, by1, 0.0); rx1 = jnp.where(valid, bx1, 0.0)
        ry2 = jnp.where(valid, by2, 0.0); rx2 = jnp.where(valid, bx2, 0.0)
        koh = jnp.where(flat3 == k.astype(f32), 1.0, 0.0)
        ny1 = ny1 + koh * ry1; nx1 = nx1 + koh * rx1
        ny2 = ny2 + koh * ry2; nx2 = nx2 + koh * rx2
        iy1 = jnp.maximum(y1, by1); ix1 = jnp.maximum(x1, bx1)
        iy2 = jnp.minimum(y2, by2); ix2 = jnp.minimum(x2, bx2)
        inter = jnp.maximum(iy2 - iy1, 0.0) * jnp.maximum(ix2 - ix1, 0.0)
        union = area + barea - inter
        iou = inter / jnp.maximum(union, 1e-8)
        sc = jnp.where(iou > 0.5, _NEG_INF, sc)
        return sc, ny1, nx1, ny2, nx2

    z3 = jnp.zeros((_R3, 128), f32)
    _, ny1, nx1, ny2, nx2 = lax.fori_loop(
        0, _NMS, _nms_body, (sc, z3, z3, z3, z3), unroll=2)

    narea = jnp.maximum(ny2 - ny1, 0.0) * jnp.maximum(nx2 - nx1, 0.0)
    max_iou = jnp.full((_R3, 128), -1e30, f32)
    amax = jnp.zeros((_R3, 128), f32)
    for g in range(n_gt):
        gy1 = gtb_ref[0, g, 0]; gx1 = gtb_ref[0, g, 1]
        gy2 = gtb_ref[0, g, 2]; gx2 = gtb_ref[0, g, 3]
        garea = (jnp.maximum(gy2 - gy1, 0.0) * jnp.maximum(gx2 - gx1, 0.0))
        iy1 = jnp.maximum(ny1, gy1); ix1 = jnp.maximum(nx1, gx1)
        iy2 = jnp.minimum(ny2, gy2); ix2 = jnp.minimum(nx2, gx2)
        inter = jnp.maximum(iy2 - iy1, 0.0) * jnp.maximum(ix2 - ix1, 0.0)
        iou_g = inter / jnp.maximum(narea + garea - inter, 1e-8)
        upd = iou_g > max_iou
        amax = jnp.where(upd, float(g), amax)
        max_iou = jnp.where(upd, iou_g, max_iou)

    valid3 = flat3 < float(_NMS)
    posc = jnp.where(valid3, max_iou, -1e30)
    negc = jnp.where(valid3, -max_iou, -1e30)

    lane = lax.broadcasted_iota(jnp.int32, (1, 128), 1).astype(f32)

    def _sel_body(k, carry):
        posc, negc, ry1, rx1, ry2, rx2, gx = carry
        kf = k.astype(f32)
        use_pos = k < _P
        cur = jnp.where(use_pos, posc, negc)
        m = jnp.max(cur)
        idx = jnp.min(jnp.where(cur == m, flat3, 1e9))
        oh = (flat3 == idx)
        ohf = jnp.where(oh, 1.0, 0.0)
        posc = jnp.where(oh & use_pos, -1e30, posc)
        negc = jnp.where(oh & jnp.logical_not(use_pos), -1e30, negc)
        by1 = jnp.sum(ohf * ny1); bx1 = jnp.sum(ohf * nx1)
        by2 = jnp.sum(ohf * ny2); bx2 = jnp.sum(ohf * nx2)
        bg = jnp.sum(ohf * amax)
        koh = jnp.where(lane == kf, 1.0, 0.0)
        ry1 = ry1 + koh * by1; rx1 = rx1 + koh * bx1
        ry2 = ry2 + koh * by2; rx2 = rx2 + koh * bx2
        gx = gx + koh * bg
        return posc, negc, ry1, rx1, ry2, rx2, gx

    zl = jnp.zeros((1, 128), f32)
    _, _, ry1, rx1, ry2, rx2, gx = lax.fori_loop(
        0, _T, _sel_body, (posc, negc, zl, zl, zl, zl, zl))

    ispos = lane < float(_P)
    gty1 = zl; gtx1 = zl; gty2 = zl; gtx2 = zl
    labv = jnp.full((1, 128), float(_L - 1), f32)
    for g in range(n_gt):
        selg = ispos & (gx == float(g))
        gty1 = jnp.where(selg, gtb_ref[0, g, 0], gty1)
        gtx1 = jnp.where(selg, gtb_ref[0, g, 1], gtx1)
        gty2 = jnp.where(selg, gtb_ref[0, g, 2], gty2)
        gtx2 = jnp.where(selg, gtb_ref[0, g, 3], gtx2)
        labv = jnp.where(selg, gtl_ref[0, 0, g].astype(f32), labv)

    bw = rx2 - rx1
    bh = ry2 - ry1
    bcx = rx1 + 0.5 * bw
    bcy = ry1 + 0.5 * bh
    gw = gtx2 - gtx1
    gh = gty2 - gty1
    gcx = gtx1 + 0.5 * gw
    gcy = gty1 + 0.5 * gh
    bw_s = jnp.where(bw <= 0, 1e-3, bw)
    bh_s = jnp.where(bh <= 0, 1e-3, bh)
    gw_s = jnp.where(gw <= 0, 1.0, gw)
    gh_s = jnp.where(gh <= 0, 1.0, gh)
    dx = jnp.where(gw == 0, 0.0, (gcx - bcx) / bw_s)
    dy = jnp.where(gh == 0, 0.0, (gcy - bcy) / bh_s)
    dw = jnp.where(gw == 0, 0.0, jnp.log(gw_s / bw_s))
    dh = jnp.where(gh == 0, 0.0, jnp.log(gh_s / bh_s))

    roi_ref[0, 0] = jnp.reshape(ry1, (128,))
    roi_ref[0, 1] = jnp.reshape(rx1, (128,))
    roi_ref[0, 2] = jnp.reshape(ry2, (128,))
    roi_ref[0, 3] = jnp.reshape(rx2, (128,))

    lab_i = labv.astype(jnp.int32)
    for l in range(_L):
        ohl = (lab_i == l)
        ohlf = jnp.where(ohl, 1.0, 0.0)
        dout_ref[0, 4 * l + 0] = jnp.reshape(ohlf * dy, (128,))
        dout_ref[0, 4 * l + 1] = jnp.reshape(ohlf * dx, (128,))
        dout_ref[0, 4 * l + 2] = jnp.reshape(ohlf * dh, (128,))
        dout_ref[0, 4 * l + 3] = jnp.reshape(ohlf * dw, (128,))
        lout_ref[0, l] = jnp.reshape(ohl.astype(jnp.int32), (128,))


def _crop_kernel(fm_ref, roi_ref, out_ref, *, H, W):
    f32 = jnp.float32

    def body(t, _):
        by1 = roi_ref[0, t, 0]
        bx1 = roi_ref[0, t, 1]
        by2 = roi_ref[0, t, 2]
        bx2 = roi_ref[0, t, 3]
        ystep = (by2 - by1) * (float(H - 1) / float(_CH - 1))
        xstep = (bx2 - bx1) * (float(W - 1) / float(_CW - 1))
        for k in range(_CH):
            ys = by1 * float(H - 1) + float(k) * ystep
            y0 = jnp.floor(ys)
            y0i = jnp.clip(y0, 0.0, float(H - 1)).astype(jnp.int32)
            y1i = jnp.clip(y0 + 1.0, 0.0, float(H - 1)).astype(jnp.int32)
            wy = ys - y0
            vy = jnp.logical_and(ys >= 0.0, ys <= float(H - 1))
            for l in range(_CW):
                xs = bx1 * float(W - 1) + float(l) * xstep
                x0 = jnp.floor(xs)
                x0i = jnp.clip(x0, 0.0, float(W - 1)).astype(jnp.int32)
                x1i = jnp.clip(x0 + 1.0, 0.0, float(W - 1)).astype(jnp.int32)
                wx = xs - x0
                vx = jnp.logical_and(xs >= 0.0, xs <= float(W - 1))
                v00 = fm_ref[0, y0i, x0i, :]
                v01 = fm_ref[0, y0i, x1i, :]
                v10 = fm_ref[0, y1i, x0i, :]
                v11 = fm_ref[0, y1i, x1i, :]
                top = v00 * (1.0 - wx) + v01 * wx
                bot = v10 * (1.0 - wx) + v11 * wx
                o = top * (1.0 - wy) + bot * wy
                ok = jnp.logical_and(vy, vx)
                o = jnp.where(ok, o, jnp.zeros_like(o)).astype(f32)
                out_ref[0, t, k, l, :] = o
        return 0

    lax.fori_loop(0, _T, body, 0)


def kernel(feature_map, rpn_bbox_deltas, rpn_labels, anchors, gt_boxes,
           gt_labels):
    B, N = anchors.shape[0], anchors.shape[1]
    H, W, C = feature_map.shape[1], feature_map.shape[2], feature_map.shape[3]
    n_gt = gt_boxes.shape[1]
    f32 = jnp.float32

    scores = rpn_labels.reshape(B, N)
    pad = _NPAD - N
    scores_p = jnp.pad(scores, ((0, 0), (0, pad)),
                       constant_values=-1.0).reshape(B, _R, 128)
    anch_p = jnp.pad(anchors, ((0, 0), (0, pad), (0, 0)))
    anch_p = anch_p.transpose(0, 2, 1).reshape(B, 4, _R, 128)
    delt_p = jnp.pad(rpn_bbox_deltas.reshape(B, N, 4), ((0, 0), (0, pad), (0, 0)))
    delt_p = delt_p.transpose(0, 2, 1).reshape(B, 4, _R, 128)

    sel = pl.pallas_call(
        functools.partial(_sel_kernel, n_real=N, n_gt=n_gt),
        grid=(B,),
        in_specs=[
            pl.BlockSpec((1, _R, 128), lambda b: (b, 0, 0)),
            pl.BlockSpec((1, 4, _R, 128), lambda b: (b, 0, 0, 0)),
            pl.BlockSpec((1, 4, _R, 128), lambda b: (b, 0, 0, 0)),
            pl.BlockSpec((1, n_gt, 4), lambda b: (b, 0, 0),
                         memory_space=pltpu.SMEM),
            pl.BlockSpec((1, 1, n_gt), lambda b: (b, 0, 0),
                         memory_space=pltpu.SMEM),
        ],
        out_specs=[
            pl.BlockSpec((1, 4, 128), lambda b: (b, 0, 0)),
            pl.BlockSpec((1, 4 * _L, 128), lambda b: (b, 0, 0)),
            pl.BlockSpec((1, _L, 128), lambda b: (b, 0, 0)),
        ],
        out_shape=[
            jax.ShapeDtypeStruct((B, 4, 128), f32),
            jax.ShapeDtypeStruct((B, 4 * _L, 128), f32),
            jax.ShapeDtypeStruct((B, _L, 128), jnp.int32),
        ],
    )(scores_p, anch_p, delt_p, gt_boxes, gt_labels.reshape(B, 1, n_gt))
    roi_t, dout_t, lout_t = sel

    roi_rows = roi_t.transpose(0, 2, 1)

    final = pl.pallas_call(
        functools.partial(_crop_kernel, H=H, W=W),
        grid=(B,),
        in_specs=[
            pl.BlockSpec((1, H, W, C), lambda b: (b, 0, 0, 0)),
            pl.BlockSpec((1, _T, 4), lambda b: (b, 0, 0),
                         memory_space=pltpu.SMEM),
        ],
        out_specs=pl.BlockSpec((1, _T, _CH, _CW, C),
                               lambda b: (b, 0, 0, 0, 0)),
        out_shape=jax.ShapeDtypeStruct((B, _T, _CH, _CW, C), f32),
    )(feature_map, roi_rows)

    roi_bbox_deltas_out = dout_t.transpose(0, 2, 1)
    roi_bbox_labels = lout_t.transpose(0, 2, 1)
    return (final, lax.stop_gradient(roi_bbox_deltas_out), roi_bbox_labels)
